# Initial kernel scaffold; baseline (speedup 1.0000x reference)
import jax
import jax.numpy as jnp
from jax.experimental import pallas as pl


def kernel(inputs, tables):
    raise NotImplementedError("write your pallas kernel here")


# speedup vs baseline: 13.6236x; 13.6236x over previous
import functools

import jax
import jax.numpy as jnp
from jax import lax
from jax.experimental import pallas as pl
from jax.experimental.pallas import tpu as pltpu
from jax.experimental.pallas import tpu_sc as plsc

B = 16384
F = 26
V1 = 51
D = 32

NC = 2
NS = 16
L = 16
NW = NC * NS

N = B * F
NPW = N // NW
CHUNK = 128
NCH = NPW // CHUNK

_mesh = plsc.VectorSubcoreMesh(core_axis_name="c", subcore_axis_name="s")


@functools.partial(
    pl.kernel,
    mesh=_mesh,
    out_type=jax.ShapeDtypeStruct((N, D), jnp.float32),
    compiler_params=pltpu.CompilerParams(use_tc_tiling_on_sc=False),
    scratch_types=[
        pltpu.VMEM((NCH, CHUNK), jnp.int32),
        pltpu.VMEM((CHUNK,), jnp.int32),
        pltpu.VMEM((CHUNK, D), jnp.float32),
        pltpu.SemaphoreType.DMA,
    ],
)
def _gather_kernel(idx_hbm, tbl_hbm, out_hbm, idx_raw, fidx, rows, sem):
    wid = lax.axis_index("s") * NC + lax.axis_index("c")
    base = wid * NPW
    pltpu.sync_copy(idx_hbm.at[wid], idx_raw)

    lane = lax.iota(jnp.int32, L)

    def chunk_body(c, carry):
        for j in range(CHUNK // L):
            pos = c * CHUNK + (j * L) + lane
            f = lax.rem(pos, F)
            fidx[pl.ds(j * L, L)] = idx_raw[c, pl.ds(j * L, L)] + f * V1
        pltpu.async_copy(tbl_hbm.at[fidx], rows, sem).wait()
        pltpu.sync_copy(rows, out_hbm.at[pl.ds(base + c * CHUNK, CHUNK)])
        return carry

    lax.fori_loop(0, NCH, chunk_body, None)


def kernel(inputs, tables):
    idx2d = inputs.reshape(NW, NCH, CHUNK)
    tbl = tables.reshape(F * V1, D)
    out = _gather_kernel(idx2d, tbl)
    return out.reshape(B, F, D)


# speedup vs baseline: 14.9188x; 1.0951x over previous
import functools

import jax
import jax.numpy as jnp
from jax import lax
from jax.experimental import pallas as pl
from jax.experimental.pallas import tpu as pltpu
from jax.experimental.pallas import tpu_sc as plsc

B = 16384
F = 26
V1 = 51
D = 32

NC = 2
NS = 16
L = 16
NW = NC * NS

N = B * F
NPW = N // NW
CHUNK = 128
NCH = NPW // CHUNK
PER = 13
G = 4
GROWS = G * CHUNK
NG = NCH // G

_mesh = plsc.VectorSubcoreMesh(core_axis_name="c", subcore_axis_name="s")


@functools.partial(
    pl.kernel,
    mesh=_mesh,
    out_type=jax.ShapeDtypeStruct((N, D), jnp.float32),
    compiler_params=pltpu.CompilerParams(use_tc_tiling_on_sc=False),
    scratch_types=[
        pltpu.VMEM((NCH, CHUNK), jnp.int32),
        pltpu.VMEM((NCH, CHUNK), jnp.int32),
        pltpu.VMEM((PER, CHUNK), jnp.int32),
        pltpu.VMEM((GROWS, D), jnp.float32),
        pltpu.VMEM((GROWS, D), jnp.float32),
        pltpu.SemaphoreType.DMA,
        pltpu.SemaphoreType.DMA,
        pltpu.SemaphoreType.DMA,
        pltpu.SemaphoreType.DMA,
    ],
)
def _gather_kernel(idx_hbm, tbl_hbm, out_hbm, idx_raw, fidx, offs,
                   rows0, rows1, gsem0, gsem1, osem0, osem1):
    wid = lax.axis_index("s") * NC + lax.axis_index("c")
    base = wid * NPW
    pltpu.sync_copy(idx_hbm.at[wid], idx_raw)

    lane = lax.iota(jnp.int32, L)

    def off_body(p, carry):
        for j in range(CHUNK // L):
            pos = p * CHUNK + (j * L) + lane
            offs[p, pl.ds(j * L, L)] = lax.rem(pos, F) * V1
        return carry

    lax.fori_loop(0, PER, off_body, None)

    def fid_body(c, carry):
        p = lax.rem(c, PER)
        for j in range(CHUNK // L):
            s = pl.ds(j * L, L)
            fidx[c, s] = idx_raw[c, s] + offs[p, s]
        return carry

    lax.fori_loop(0, NCH, fid_body, None)

    rows = (rows0, rows1)
    gsem = (gsem0, gsem1)
    osem = (osem0, osem1)

    def issue_g(g, b):
        for k in range(G):
            pltpu.async_copy(tbl_hbm.at[fidx.at[g * G + k]],
                             rows[b].at[pl.ds(k * CHUNK, CHUNK)], gsem[b])

    def drain_g(b):
        pltpu.make_async_copy(tbl_hbm.at[pl.ds(0, GROWS)], rows[b], gsem[b]).wait()

    def issue_o(g, b):
        pltpu.async_copy(rows[b], out_hbm.at[pl.ds(base + g * GROWS, GROWS)],
                         osem[b])

    def drain_o(b):
        pltpu.make_async_copy(rows[b], out_hbm.at[pl.ds(base, GROWS)],
                              osem[b]).wait()

    issue_g(0, 0)
    issue_g(1, 1)

    def pipe_body(i, carry):
        g0 = 2 * i
        drain_g(0)
        issue_o(g0, 0)
        drain_g(1)
        issue_o(g0 + 1, 1)

        @pl.when(i < NG // 2 - 1)
        def _more():
            drain_o(0)
            issue_g(g0 + 2, 0)
            drain_o(1)
            issue_g(g0 + 3, 1)

        return carry

    lax.fori_loop(0, NG // 2, pipe_body, None)
    drain_o(0)
    drain_o(1)


def kernel(inputs, tables):
    idx2d = inputs.reshape(NW, NCH, CHUNK)
    tbl = tables.reshape(F * V1, D)
    out = _gather_kernel(idx2d, tbl)
    return out.reshape(B, F, D)


# speedup vs baseline: 38.0373x; 2.5496x over previous
import functools

import jax
import jax.numpy as jnp
from jax import lax
from jax.experimental import pallas as pl
from jax.experimental.pallas import tpu as pltpu
from jax.experimental.pallas import tpu_sc as plsc

B = 16384
F = 26
V1 = 51
D = 32

NC = 2
NS = 16
L = 16
NW = NC * NS

BPW = B // NW
DT = D // 8
BT = B // 128
BTW = BPW // 128
TT = F * D * V1

_mesh = plsc.VectorSubcoreMesh(core_axis_name="c", subcore_axis_name="s")


@functools.partial(
    pl.kernel,
    mesh=_mesh,
    out_type=jax.ShapeDtypeStruct((F, DT, BT, 8, 128), jnp.float32),
    compiler_params=pltpu.CompilerParams(use_tc_tiling_on_sc=False,
                                         needs_layout_passes=False),
    scratch_types=[
        pltpu.VMEM((TT,), jnp.float32),
        pltpu.VMEM((F, BPW), jnp.int32),
        pltpu.VMEM((DT, BTW, 8, 128), jnp.float32),
        pltpu.VMEM((DT, BTW, 8, 128), jnp.float32),
        pltpu.SemaphoreType.DMA,
        pltpu.SemaphoreType.DMA,
    ],
)
def _gather_kernel(idx_hbm, tt_hbm, out_hbm, tt_v, idx_v, blk0, blk1,
                   osem0, osem1):
    wid = lax.axis_index("s") * NC + lax.axis_index("c")
    pltpu.sync_copy(tt_hbm, tt_v)
    pltpu.sync_copy(idx_hbm.at[:, wid], idx_v)

    blks = (blk0, blk1)
    sems = (osem0, osem1)

    def fill_f(f, blk):
        def bt_body(bt, carry):
            idxvs = [idx_v[f, pl.ds(bt * 128 + j * L, L)] for j in range(8)]
            base0 = f * (D * V1)
            for dt in range(DT):
                for di in range(8):
                    base = base0 + (dt * 8 + di) * V1
                    for j in range(8):
                        vals = plsc.load_gather(tt_v, [idxvs[j] + base])
                        blk[dt, bt, di, pl.ds(j * L, L)] = vals
            return carry

        lax.fori_loop(0, BTW, bt_body, None)

    def issue_out(f, b):
        for dt in range(DT):
            pltpu.async_copy(blks[b].at[dt],
                             out_hbm.at[f, dt, pl.ds(wid * BTW, BTW)],
                             sems[b])

    def drain_out(b):
        for dt in range(DT):
            pltpu.make_async_copy(blks[b].at[dt],
                                  out_hbm.at[0, dt, pl.ds(wid * BTW, BTW)],
                                  sems[b]).wait()

    def f_loop(i, carry):
        for b in range(2):
            f = 2 * i + b

            @pl.when(i > 0)
            def _reuse():
                drain_out(b)

            fill_f(f, blks[b])
            issue_out(f, b)
        return carry

    lax.fori_loop(0, F // 2, f_loop, None)
    drain_out(0)
    drain_out(1)


def kernel(inputs, tables):
    idx3 = inputs.T.reshape(F, NW, BPW)
    tt1 = tables.transpose(0, 2, 1).reshape(TT)
    out5 = _gather_kernel(idx3, tt1)
    return out5.transpose(2, 4, 0, 1, 3).reshape(B, F, D)


# speedup vs baseline: 95.3920x; 2.5079x over previous
import functools

import jax
import jax.numpy as jnp
from jax import lax
from jax.experimental import pallas as pl
from jax.experimental.pallas import tpu as pltpu
from jax.experimental.pallas import tpu_sc as plsc

B = 16384
F = 26
V1 = 51
D = 32

NC = 2
NS = 16
L = 16
NW = NC * NS

BPW = B // NW
DT = D // 8
BT = B // 128
BTW = BPW // 128
TT = F * D * V1

_mesh = plsc.VectorSubcoreMesh(core_axis_name="c", subcore_axis_name="s")


@functools.partial(
    pl.kernel,
    mesh=_mesh,
    out_type=jax.ShapeDtypeStruct((F, DT, BT, 8, 128), jnp.float32),
    compiler_params=pltpu.CompilerParams(use_tc_tiling_on_sc=False,
                                         needs_layout_passes=False),
    scratch_types=[
        pltpu.VMEM((TT,), jnp.float32),
        pltpu.VMEM((F, BPW), jnp.int32),
        pltpu.VMEM((DT, BTW, 8, 128), jnp.float32),
        pltpu.VMEM((DT, BTW, 8, 128), jnp.float32),
        pltpu.SemaphoreType.DMA,
        pltpu.SemaphoreType.DMA,
    ],
)
def _gather_kernel(idx_hbm, tt_hbm, out_hbm, tt_v, idx_v, blk0, blk1,
                   osem0, osem1):
    wid = lax.axis_index("s") * NC + lax.axis_index("c")
    pltpu.sync_copy(tt_hbm, tt_v)
    pltpu.sync_copy(idx_hbm.at[:, wid], idx_v)

    blks = (blk0, blk1)
    sems = (osem0, osem1)

    def fill_f(f, blk):
        base0 = f * (D * V1)

        @plsc.parallel_loop(0, BTW * 8, unroll=2)
        def _grp(k):
            bt = k // 8
            j = lax.rem(k, 8)
            idxv = idx_v[f, pl.ds(bt * 128 + j * L, L)]
            for dt in range(DT):
                for di in range(8):
                    base = base0 + (dt * 8 + di) * V1
                    vals = plsc.load_gather(tt_v, [idxv + base])
                    blk[dt, bt, di, pl.ds(j * L, L)] = vals

    def issue_out(f, b):
        for dt in range(DT):
            pltpu.async_copy(blks[b].at[dt],
                             out_hbm.at[f, dt, pl.ds(wid * BTW, BTW)],
                             sems[b])

    def drain_out(b):
        for dt in range(DT):
            pltpu.make_async_copy(blks[b].at[dt],
                                  out_hbm.at[0, dt, pl.ds(wid * BTW, BTW)],
                                  sems[b]).wait()

    def f_loop(i, carry):
        for b in range(2):
            f = 2 * i + b

            @pl.when(i > 0)
            def _reuse():
                drain_out(b)

            fill_f(f, blks[b])
            issue_out(f, b)
        return carry

    lax.fori_loop(0, F // 2, f_loop, None)
    drain_out(0)
    drain_out(1)


def kernel(inputs, tables):
    idx3 = inputs.T.reshape(F, NW, BPW)
    tt1 = tables.transpose(0, 2, 1).reshape(TT)
    out5 = _gather_kernel(idx3, tt1)
    return out5.transpose(2, 4, 0, 1, 3).reshape(B, F, D)
